# Initial kernel scaffold; baseline (speedup 1.0000x reference)
import jax
import jax.numpy as jnp
from jax.experimental import pallas as pl


def kernel(inputs, edge_index, W1, b1, W2, b2, W3, b3):
    raise NotImplementedError("write your pallas kernel here")


# speedup vs baseline: 4.4618x; 4.4618x over previous
import functools

import jax
import jax.numpy as jnp
from jax import lax
from jax.experimental import pallas as pl
from jax.experimental.pallas import tpu as pltpu
from jax.experimental.pallas import tpu_sc as plsc

NC = 2
NS = 16
NW = NC * NS
K = 80

f32 = jnp.float32


def _sc_mesh():
    return plsc.VectorSubcoreMesh(
        core_axis_name="c", subcore_axis_name="s", num_cores=NC,
        num_subcores=NS)


def _make_deg_call(E, Npad):
    EPT = E // NW
    CH = EPT // K
    COLS = Npad // NS

    @functools.partial(
        pl.kernel,
        mesh=_sc_mesh(),
        out_type=jax.ShapeDtypeStruct((NC, 2, Npad), f32),
        scratch_types=[
            pltpu.VMEM((K,), jnp.int32),
            pltpu.VMEM((K,), f32),
            pltpu.VMEM((COLS,), f32),
            pltpu.VMEM_SHARED((Npad,), f32),
            pltpu.VMEM_SHARED((Npad,), f32),
        ],
    )
    def deg(src_hbm, dst_hbm, out_hbm, idx, ones_v, zbuf, dego_sh, degi_sh):
        c = lax.axis_index("c")
        s = lax.axis_index("s")
        wid = c * NS + s
        zero16 = jnp.zeros((16,), f32)
        one16 = jnp.ones((16,), f32)
        for i in range(K // 16):
            ones_v[pl.ds(16 * i, 16)] = one16
        for i in range(COLS // 16):
            zbuf[pl.ds(16 * i, 16)] = zero16
        pltpu.sync_copy(zbuf, dego_sh.at[pl.ds(s * COLS, COLS)])
        pltpu.sync_copy(zbuf, degi_sh.at[pl.ds(s * COLS, COLS)])
        plsc.subcore_barrier()

        def body(ch, carry):
            base = wid * EPT + ch * K
            pltpu.sync_copy(src_hbm.at[pl.ds(base, K)], idx)
            pltpu.sync_copy(ones_v, dego_sh.at[idx], add=True)
            pltpu.sync_copy(dst_hbm.at[pl.ds(base, K)], idx)
            pltpu.sync_copy(ones_v, degi_sh.at[idx], add=True)
            return carry

        lax.fori_loop(0, CH, body, 0)
        plsc.subcore_barrier()
        pltpu.sync_copy(dego_sh.at[pl.ds(s * COLS, COLS)],
                        out_hbm.at[c, 0, pl.ds(s * COLS, COLS)])
        pltpu.sync_copy(degi_sh.at[pl.ds(s * COLS, COLS)],
                        out_hbm.at[c, 1, pl.ds(s * COLS, COLS)])

    return deg


def _make_msg_call(E, Npad, D):
    EPT = E // NW
    CH = EPT // K
    RPT = Npad // NS

    @functools.partial(
        pl.kernel,
        mesh=_sc_mesh(),
        out_type=jax.ShapeDtypeStruct((NC, Npad, D), f32),
        scratch_types=[
            pltpu.VMEM((K,), jnp.int32),
            pltpu.VMEM((K,), jnp.int32),
            pltpu.VMEM((K, D), f32),
            pltpu.VMEM_SHARED((Npad, D), f32),
            pltpu.SemaphoreType.DMA,
        ],
    )
    def msg(t_hbm, src_hbm, dst_hbm, out_hbm, idx_s, idx_d, rows, acc, sem):
        c = lax.axis_index("c")
        s = lax.axis_index("s")
        wid = c * NS + s
        zero16 = jnp.zeros((16,), f32)
        for r in range(K):
            for j in range(D // 16):
                rows[r, pl.ds(16 * j, 16)] = zero16
        for j in range(RPT // K):
            pltpu.sync_copy(rows, acc.at[pl.ds(s * RPT + j * K, K)])
        plsc.subcore_barrier()

        def body(ch, carry):
            base = wid * EPT + ch * K
            pltpu.sync_copy(src_hbm.at[pl.ds(base, K)], idx_s)
            pltpu.sync_copy(dst_hbm.at[pl.ds(base, K)], idx_d)
            pltpu.async_copy(t_hbm.at[idx_s], rows, sem).wait()
            pltpu.sync_copy(rows, acc.at[idx_d], add=True)
            return carry

        lax.fori_loop(0, CH, body, 0)
        plsc.subcore_barrier()
        pltpu.sync_copy(acc.at[pl.ds(s * RPT, RPT)],
                        out_hbm.at[c, pl.ds(s * RPT, RPT)])

    return msg


def _prep_body(Npad, D, degp_ref, x_ref, ns_ref, nd_ref, t0_ref):
    dego = degp_ref[0, 0, :] + degp_ref[1, 0, :]
    degi = degp_ref[0, 1, :] + degp_ref[1, 1, :]
    ns = jnp.where(dego > 0, lax.rsqrt(dego), 0.0)
    nd = jnp.where(degi > 0, lax.rsqrt(degi), 0.0)
    nsb = jnp.broadcast_to(ns[:, None], (Npad, D))
    ndb = jnp.broadcast_to(nd[:, None], (Npad, D))
    ns_ref[...] = nsb
    nd_ref[...] = ndb
    t0_ref[...] = x_ref[...] * nsb


def _make_prep_call(Npad, D):
    sds = jax.ShapeDtypeStruct((Npad, D), f32)
    return pl.pallas_call(
        functools.partial(_prep_body, Npad, D),
        out_shape=[sds, sds, sds],
    )


def _layer_scaled_body(p_ref, nd_ref, ns_ref, w_ref, b_ref, o_ref):
    m = (p_ref[0] + p_ref[1]) * nd_ref[...]
    z = lax.dot_general(m, w_ref[...], (((1,), (0,)), ((), ())),
                        precision=lax.Precision.HIGHEST,
                        preferred_element_type=f32) + b_ref[...]
    o_ref[...] = z * ns_ref[...]


def _layer_final_body(p_ref, nd_ref, w_ref, b_ref, o_ref):
    m = (p_ref[0] + p_ref[1]) * nd_ref[...]
    z = lax.dot_general(m, w_ref[...], (((1,), (0,)), ((), ())),
                        precision=lax.Precision.HIGHEST,
                        preferred_element_type=f32) + b_ref[...]
    o_ref[...] = z


def _make_layer_call(Npad, D, H, scaled):
    BR = 1024
    grid = (Npad // BR,)
    row_spec = pl.BlockSpec((BR, D), lambda i: (i, 0))
    p_spec = pl.BlockSpec((NC, BR, D), lambda i: (0, i, 0))
    w_spec = pl.BlockSpec((D, H), lambda i: (0, 0))
    b_spec = pl.BlockSpec((1, H), lambda i: (0, 0))
    in_specs = [p_spec, row_spec] + ([row_spec] if scaled else []) + \
        [w_spec, b_spec]
    return pl.pallas_call(
        _layer_scaled_body if scaled else _layer_final_body,
        grid=grid,
        in_specs=in_specs,
        out_specs=pl.BlockSpec((BR, H), lambda i: (i, 0)),
        out_shape=jax.ShapeDtypeStruct((Npad, H), f32),
    )


def kernel(inputs, edge_index, W1, b1, W2, b2, W3, b3):
    N, D = inputs.shape
    H = W1.shape[1]
    E = edge_index.shape[1]
    Npad = ((N + 8 * NW - 1) // (8 * NW)) * (8 * NW)

    src = edge_index[0]
    dst = edge_index[1]
    xp = jnp.zeros((Npad, D), f32).at[:N].set(inputs)

    deg_call = _make_deg_call(E, Npad)
    msg_call = _make_msg_call(E, Npad, D)
    prep_call = _make_prep_call(Npad, D)
    layer_scaled = _make_layer_call(Npad, D, H, scaled=True)
    layer_final = _make_layer_call(Npad, D, H, scaled=False)

    degp = deg_call(src, dst)
    ns_b, nd_b, t0 = prep_call(degp, xp)

    p1 = msg_call(t0, src, dst)
    t1 = layer_scaled(p1, nd_b, ns_b, W1, b1.reshape(1, H))
    p2 = msg_call(t1, src, dst)
    t2 = layer_scaled(p2, nd_b, ns_b, W2, b2.reshape(1, H))
    p3 = msg_call(t2, src, dst)
    h = layer_final(p3, nd_b, W3, b3.reshape(1, H))

    return h[:N][None, :, :]


# speedup vs baseline: 13.2726x; 2.9747x over previous
import functools

import jax
import jax.numpy as jnp
from jax import lax
from jax.experimental import pallas as pl
from jax.experimental.pallas import tpu as pltpu
from jax.experimental.pallas import tpu_sc as plsc

NC = 2
NS = 16
NW = NC * NS
K = 80

f32 = jnp.float32


def _sc_mesh():
    return plsc.VectorSubcoreMesh(
        core_axis_name="c", subcore_axis_name="s", num_cores=NC,
        num_subcores=NS)


def _make_deg_call(E, Npad):
    EPT = E // NW
    CH = EPT // K
    COLS = Npad // NS

    @functools.partial(
        pl.kernel,
        mesh=_sc_mesh(),
        out_type=jax.ShapeDtypeStruct((NC, 2, Npad), f32),
        scratch_types=[
            pltpu.VMEM((CH, 1, K), jnp.int32),
            pltpu.VMEM((CH, 1, K), jnp.int32),
            pltpu.VMEM((K,), f32),
            pltpu.VMEM((COLS,), f32),
            pltpu.VMEM_SHARED((Npad,), f32),
            pltpu.VMEM_SHARED((Npad,), f32),
            pltpu.SemaphoreType.DMA,
        ],
    )
    def deg(src2d_hbm, dst2d_hbm, out_hbm, src2d, dst2d, ones_v, zbuf,
            dego_sh, degi_sh, sem):
        c = lax.axis_index("c")
        s = lax.axis_index("s")
        wid = c * NS + s
        zero16 = jnp.zeros((16,), f32)
        one16 = jnp.ones((16,), f32)
        for i in range(K // 16):
            ones_v[pl.ds(16 * i, 16)] = one16
        for i in range(COLS // 16):
            zbuf[pl.ds(16 * i, 16)] = zero16
        pltpu.sync_copy(zbuf, dego_sh.at[pl.ds(s * COLS, COLS)])
        pltpu.sync_copy(zbuf, degi_sh.at[pl.ds(s * COLS, COLS)])
        pltpu.sync_copy(src2d_hbm.at[pl.ds(wid * CH, CH)], src2d)
        pltpu.sync_copy(dst2d_hbm.at[pl.ds(wid * CH, CH)], dst2d)
        plsc.subcore_barrier()

        def fire(ch, carry):
            pltpu.async_copy(ones_v, dego_sh.at[src2d.at[ch, 0]], sem, add=True)
            pltpu.async_copy(ones_v, degi_sh.at[dst2d.at[ch, 0]], sem, add=True)
            return carry

        def drain(ch, carry):
            pltpu.make_async_copy(ones_v, dego_sh.at[src2d.at[ch, 0]],
                                  sem).wait()
            pltpu.make_async_copy(ones_v, degi_sh.at[dst2d.at[ch, 0]],
                                  sem).wait()
            return carry

        lax.fori_loop(0, CH, fire, 0)
        lax.fori_loop(0, CH, drain, 0)
        plsc.subcore_barrier()
        pltpu.sync_copy(dego_sh.at[pl.ds(s * COLS, COLS)],
                        out_hbm.at[c, 0, pl.ds(s * COLS, COLS)])
        pltpu.sync_copy(degi_sh.at[pl.ds(s * COLS, COLS)],
                        out_hbm.at[c, 1, pl.ds(s * COLS, COLS)])

    return deg


def _make_msg_call(E, Npad, D):
    EPT = E // NW
    CH = EPT // K
    RPT = Npad // NS
    NB = 3
    ST = ((CH - NB) // NB) * NB

    @functools.partial(
        pl.kernel,
        mesh=_sc_mesh(),
        out_type=jax.ShapeDtypeStruct((NC, Npad, D), f32),
        scratch_types=[
            pltpu.VMEM((EPT,), jnp.int32),
            [pltpu.VMEM((1, 1, K), jnp.int32)] * NB,
            [pltpu.VMEM((K, D), f32)] * NB,
            pltpu.VMEM_SHARED((Npad, D), f32),
            [pltpu.SemaphoreType.DMA] * NB,
            [pltpu.SemaphoreType.DMA] * NB,
            [pltpu.SemaphoreType.DMA] * NB,
        ],
    )
    def msg(t_hbm, src_hbm, dst2d_hbm, out_hbm, src_all, dring, rbufs, acc,
            gsems, ssems, dsems):
        c = lax.axis_index("c")
        s = lax.axis_index("s")
        wid = c * NS + s
        ebase = wid * EPT
        cbase = wid * CH
        zero16 = jnp.zeros((16,), f32)
        r0 = rbufs[0]
        for r in range(K):
            for j in range(D // 16):
                r0[r, pl.ds(16 * j, 16)] = zero16
        for j in range(RPT // K):
            pltpu.sync_copy(r0, acc.at[pl.ds(s * RPT + j * K, K)])
        pltpu.sync_copy(src_hbm.at[pl.ds(ebase, EPT)], src_all)
        plsc.subcore_barrier()

        def fetch(ch, b):
            pltpu.async_copy(dst2d_hbm.at[pl.ds(cbase + ch, 1)], dring[b],
                             dsems[b])
            pltpu.async_copy(
                t_hbm.at[src_all.at[pl.ds(ch * K, K)]], rbufs[b], gsems[b])

        def wait_fetch(b):
            pltpu.make_async_copy(dst2d_hbm.at[pl.ds(0, 1)], dring[b],
                                  dsems[b]).wait()
            pltpu.make_async_copy(
                t_hbm.at[src_all.at[pl.ds(0, K)]], rbufs[b],
                gsems[b]).wait()

        def scatter(b):
            pltpu.async_copy(rbufs[b], acc.at[dring[b].at[0, 0]], ssems[b],
                             add=True)

        def wait_scatter(b):
            pltpu.make_async_copy(rbufs[b], acc.at[dring[b].at[0, 0]],
                                  ssems[b]).wait()

        for b in range(NB):
            fetch(b, b)

        def body(gi, carry):
            for b in range(NB):
                ch = gi * NB + b
                wait_fetch(b)
                scatter(b)
                wait_scatter(b)
                fetch(ch + NB, b)
            return carry

        lax.fori_loop(0, ST // NB, body, 0)
        for ch in range(ST, CH):
            b = ch % NB
            wait_fetch(b)
            scatter(b)
            if ch + NB < CH:
                wait_scatter(b)
                fetch(ch + NB, b)
        for ch in range(max(ST, CH - NB), CH):
            wait_scatter(ch % NB)
        plsc.subcore_barrier()
        pltpu.sync_copy(acc.at[pl.ds(s * RPT, RPT)],
                        out_hbm.at[c, pl.ds(s * RPT, RPT)])

    return msg


def _prep_body(Npad, D, degp_ref, x_ref, ns_ref, nd_ref, t0_ref):
    dego = degp_ref[0, 0, :] + degp_ref[1, 0, :]
    degi = degp_ref[0, 1, :] + degp_ref[1, 1, :]
    ns = jnp.where(dego > 0, lax.rsqrt(dego), 0.0)
    nd = jnp.where(degi > 0, lax.rsqrt(degi), 0.0)
    nsb = jnp.broadcast_to(ns[:, None], (Npad, D))
    ndb = jnp.broadcast_to(nd[:, None], (Npad, D))
    ns_ref[...] = nsb
    nd_ref[...] = ndb
    t0_ref[...] = x_ref[...] * nsb


def _make_prep_call(Npad, D):
    sds = jax.ShapeDtypeStruct((Npad, D), f32)
    return pl.pallas_call(
        functools.partial(_prep_body, Npad, D),
        out_shape=[sds, sds, sds],
    )


def _layer_scaled_body(p_ref, nd_ref, ns_ref, w_ref, b_ref, o_ref):
    m = (p_ref[0] + p_ref[1]) * nd_ref[...]
    z = lax.dot_general(m, w_ref[...], (((1,), (0,)), ((), ())),
                        precision=lax.Precision.HIGHEST,
                        preferred_element_type=f32) + b_ref[...]
    o_ref[...] = z * ns_ref[...]


def _layer_final_body(p_ref, nd_ref, w_ref, b_ref, o_ref):
    m = (p_ref[0] + p_ref[1]) * nd_ref[...]
    z = lax.dot_general(m, w_ref[...], (((1,), (0,)), ((), ())),
                        precision=lax.Precision.HIGHEST,
                        preferred_element_type=f32) + b_ref[...]
    o_ref[...] = z


def _make_layer_call(Npad, D, H, scaled):
    BR = 1024
    grid = (Npad // BR,)
    row_spec = pl.BlockSpec((BR, D), lambda i: (i, 0))
    p_spec = pl.BlockSpec((NC, BR, D), lambda i: (0, i, 0))
    w_spec = pl.BlockSpec((D, H), lambda i: (0, 0))
    b_spec = pl.BlockSpec((1, H), lambda i: (0, 0))
    in_specs = [p_spec, row_spec] + ([row_spec] if scaled else []) + \
        [w_spec, b_spec]
    return pl.pallas_call(
        _layer_scaled_body if scaled else _layer_final_body,
        grid=grid,
        in_specs=in_specs,
        out_specs=pl.BlockSpec((BR, H), lambda i: (i, 0)),
        out_shape=jax.ShapeDtypeStruct((Npad, H), f32),
    )


def kernel(inputs, edge_index, W1, b1, W2, b2, W3, b3):
    N, D = inputs.shape
    H = W1.shape[1]
    E = edge_index.shape[1]
    Npad = ((N + 8 * NW - 1) // (8 * NW)) * (8 * NW)

    src = edge_index[0]
    dst = edge_index[1]
    src2d = src.reshape(E // K, 1, K)
    dst2d = dst.reshape(E // K, 1, K)
    xp = jnp.zeros((Npad, D), f32).at[:N].set(inputs)

    deg_call = _make_deg_call(E, Npad)
    msg_call = _make_msg_call(E, Npad, D)
    prep_call = _make_prep_call(Npad, D)
    layer_scaled = _make_layer_call(Npad, D, H, scaled=True)
    layer_final = _make_layer_call(Npad, D, H, scaled=False)

    degp = deg_call(src2d, dst2d)
    ns_b, nd_b, t0 = prep_call(degp, xp)

    p1 = msg_call(t0, src, dst2d)
    t1 = layer_scaled(p1, nd_b, ns_b, W1, b1.reshape(1, H))
    p2 = msg_call(t1, src, dst2d)
    t2 = layer_scaled(p2, nd_b, ns_b, W2, b2.reshape(1, H))
    p3 = msg_call(t2, src, dst2d)
    h = layer_final(p3, nd_b, W3, b3.reshape(1, H))

    return h[:N][None, :, :]
